# baseline (speedup 1.0000x reference)
import functools

import jax
import jax.numpy as jnp
from jax.experimental import pallas as pl
from jax.experimental.shard_map import shard_map
from jax.sharding import Mesh, PartitionSpec as P

G, N, K = 2048, 40, 8
C_IN, C_OUT = 32, 6
H = W = 512
D_IN = 576

GB = 64
SB = 128


def _stats_kernel(x_ref, out_ref):
    i = pl.program_id(0)
    x = x_ref[...]
    s = jnp.sum(x, axis=(0, 1))
    s2 = jnp.sum(x * x, axis=(0, 1))
    part = jnp.stack([s, s2], axis=0)

    @pl.when(i == 0)
    def _():
        out_ref[...] = jnp.zeros_like(out_ref)

    out_ref[...] += part


def _layer(xf, a_blk, w_ref, b_ref, d_in, f_out):
    xb = xf.astype(jnp.bfloat16)
    pa = jnp.dot(xb, w_ref[:d_in, :], preferred_element_type=jnp.float32)
    pb = jnp.dot(xb, w_ref[d_in:, :], preferred_element_type=jnp.float32)
    pb3 = pb.reshape(GB, N, f_out)
    agg = jax.lax.dot_general(
        a_blk, pb3.astype(jnp.bfloat16),
        dimension_numbers=(((2,), (1,)), ((0,), (0,))),
        preferred_element_type=jnp.float32)
    h = pa.reshape(GB, N, f_out) + agg + b_ref[...]
    return jnp.maximum(h, 0.0).reshape(GB * N, f_out)


def _gcn_kernel(x_ref, a_ref, knn_ref, stats_ref,
                w1_ref, b1_ref, w2_ref, b2_ref, w3_ref, b3_ref, w4_ref, b4_ref,
                wc1_ref, bc1_ref, pa_ref, wc2_ref, bc2_ref,
                out_ref):
    total = float(G * N)
    mean = stats_ref[0, :] / total
    var = stats_ref[1, :] / total - mean * mean
    rinv = jax.lax.rsqrt(var + 1e-5)

    x = (x_ref[...] - mean) * rinv
    a_blk = a_ref[...].astype(jnp.bfloat16)

    xf = x.reshape(GB * N, D_IN)
    xf = _layer(xf, a_blk, w1_ref, b1_ref, D_IN, 512)
    xf = _layer(xf, a_blk, w2_ref, b2_ref, 512, 256)
    xf = _layer(xf, a_blk, w3_ref, b3_ref, 256, 128)
    xf = _layer(xf, a_blk, w4_ref, b4_ref, 128, 64)

    h = jnp.dot(xf, wc1_ref[...], preferred_element_type=jnp.float32) + bc1_ref[...]
    h = jnp.where(h >= 0, h, pa_ref[...] * h)
    p = (jnp.dot(h, wc2_ref[...], preferred_element_type=jnp.float32)
         + bc2_ref[...])
    p3 = p.reshape(GB, N, 2)

    ids = knn_ref[...]
    iota_n = jax.lax.broadcasted_iota(jnp.int32, (GB, N), 1)
    edges = []
    for k in range(K):
        mask = (iota_n == ids[:, k][:, None]).astype(jnp.float32)
        edges.append(jnp.sum(mask[:, :, None] * p3, axis=1))
    out_ref[...] = jnp.stack(edges, axis=1).reshape(GB * K, 2)


def _conv_kernel(x_ref, w_ref, b_ref, out_ref):
    out_ref[...] = (jnp.dot(w_ref[...], x_ref[...],
                            preferred_element_type=jnp.float32) + b_ref[...])


def _local_compute(inputs, node_feats, A, knn_inds, conv_w, conv_b,
                   w1b, b1, w2b, b2, w3b, b3, w4b, b4,
                   wc1, bc1, prelu_a, wc2, bc2, axis_name):
    g_loc = node_feats.shape[0]

    stats = pl.pallas_call(
        _stats_kernel,
        grid=(g_loc // SB,),
        in_specs=[pl.BlockSpec((SB, N, D_IN), lambda i: (i, 0, 0))],
        out_specs=pl.BlockSpec((2, D_IN), lambda i: (0, 0)),
        out_shape=jax.ShapeDtypeStruct((2, D_IN), jnp.float32),
    )(node_feats)
    stats = jax.lax.psum(stats, axis_name)

    const = lambda shape: pl.BlockSpec(shape, lambda i: tuple(0 for _ in shape))
    gcn_pred = pl.pallas_call(
        _gcn_kernel,
        grid=(g_loc // GB,),
        in_specs=[
            pl.BlockSpec((GB, N, D_IN), lambda i: (i, 0, 0)),
            pl.BlockSpec((GB, N, N), lambda i: (i, 0, 0)),
            pl.BlockSpec((GB, K), lambda i: (i, 0)),
            const((2, D_IN)),
            const((2 * D_IN, 512)), const((512,)),
            const((1024, 256)), const((256,)),
            const((512, 128)), const((128,)),
            const((256, 64)), const((64,)),
            const((64, 32)), const((32,)), const((32,)),
            const((32, 2)), const((2,)),
        ],
        out_specs=pl.BlockSpec((GB * K, 2), lambda i: (i, 0)),
        out_shape=jax.ShapeDtypeStruct((g_loc * K, 2), jnp.float32),
    )(node_feats, A, knn_inds, stats,
      w1b, b1, w2b, b2, w3b, b3, w4b, b4, wc1, bc1, prelu_a, wc2, bc2)

    hw_loc = inputs.shape[2] * inputs.shape[3]
    HWB = min(16384, hw_loc)
    x2 = inputs.reshape(C_IN, hw_loc)
    pred = pl.pallas_call(
        _conv_kernel,
        grid=(hw_loc // HWB,),
        in_specs=[
            pl.BlockSpec((C_IN, HWB), lambda i: (0, i)),
            pl.BlockSpec((C_OUT, C_IN), lambda i: (0, 0)),
            pl.BlockSpec((C_OUT, 1), lambda i: (0, 0)),
        ],
        out_specs=pl.BlockSpec((C_OUT, HWB), lambda i: (0, i)),
        out_shape=jax.ShapeDtypeStruct((C_OUT, hw_loc), jnp.float32),
    )(x2, conv_w, conv_b.reshape(C_OUT, 1))
    pred_maps = pred.reshape(1, C_OUT, inputs.shape[2], inputs.shape[3])

    return pred_maps, gcn_pred


def kernel(inputs, node_feats, A, knn_inds, conv_w, conv_b,
           w1, b1, w2, b2, w3, b3, w4, b4, wc1, bc1, prelu_a, wc2, bc2):
    w1b = w1.astype(jnp.bfloat16)
    w2b = w2.astype(jnp.bfloat16)
    w3b = w3.astype(jnp.bfloat16)
    w4b = w4.astype(jnp.bfloat16)

    devs = jax.devices()
    ndev = 2 if len(devs) >= 2 and G % (2 * GB) == 0 else 1
    mesh = Mesh(devs[:ndev], ("x",))
    shard = P("x")
    rep = P()
    fn = shard_map(
        functools.partial(_local_compute, axis_name="x"),
        mesh=mesh,
        in_specs=(P(None, None, None, "x"), shard, shard, shard,
                  rep, rep, rep, rep, rep, rep, rep, rep,
                  rep, rep, rep, rep, rep, rep, rep),
        out_specs=(P(None, None, None, "x"), shard),
        check_rep=False,
    )
    return fn(inputs, node_feats, A, knn_inds, conv_w, conv_b,
              w1b, b1, w2b, b2, w3b, b3, w4b, b4,
              wc1, bc1, prelu_a, wc2, bc2)


# speedup vs baseline: 1.6042x; 1.6042x over previous
import jax
import jax.numpy as jnp
from jax.experimental import pallas as pl

G, N, K = 2048, 40, 8
C_IN, C_OUT = 32, 6
H = W = 512
D_IN = 576

GB = 64
SB = 128
HWB = (H * W) // (G // GB)


def _stats_kernel(x_ref, out_ref):
    i = pl.program_id(0)
    x = x_ref[...]
    s = jnp.sum(x, axis=(0, 1))
    s2 = jnp.sum(x * x, axis=(0, 1))
    part = jnp.stack([s, s2], axis=0)

    @pl.when(i == 0)
    def _():
        out_ref[...] = jnp.zeros_like(out_ref)

    out_ref[...] += part


def _layer(xf, a_blk, w_ref, b_ref, d_in, f_out):
    xb = xf.astype(jnp.bfloat16)
    pa = jnp.dot(xb, w_ref[:d_in, :], preferred_element_type=jnp.float32)
    pb = jnp.dot(xb, w_ref[d_in:, :], preferred_element_type=jnp.float32)
    pb3 = pb.reshape(GB, N, f_out)
    agg = jax.lax.dot_general(
        a_blk, pb3.astype(jnp.bfloat16),
        dimension_numbers=(((2,), (1,)), ((0,), (0,))),
        preferred_element_type=jnp.float32)
    h = pa.reshape(GB, N, f_out) + agg + b_ref[...]
    return jnp.maximum(h, 0.0).reshape(GB * N, f_out)


def _gcn_kernel(x_ref, a_ref, knn_ref, stats_ref,
                w1_ref, b1_ref, w2_ref, b2_ref, w3_ref, b3_ref, w4_ref, b4_ref,
                wc1_ref, bc1_ref, pa_ref, wc2_ref, bc2_ref,
                img_ref, cw_ref, cb_ref,
                out_ref, pred_ref):
    pred_ref[...] = (jnp.dot(cw_ref[...], img_ref[...],
                             preferred_element_type=jnp.float32) + cb_ref[...])

    total = float(G * N)
    mean = stats_ref[0, :] / total
    var = stats_ref[1, :] / total - mean * mean
    rinv = jax.lax.rsqrt(var + 1e-5)

    x = (x_ref[...] - mean) * rinv
    a_blk = a_ref[...].astype(jnp.bfloat16)

    xf = x.reshape(GB * N, D_IN)
    xf = _layer(xf, a_blk, w1_ref, b1_ref, D_IN, 512)
    xf = _layer(xf, a_blk, w2_ref, b2_ref, 512, 256)
    xf = _layer(xf, a_blk, w3_ref, b3_ref, 256, 128)
    xf = _layer(xf, a_blk, w4_ref, b4_ref, 128, 64)

    h = jnp.dot(xf, wc1_ref[...], preferred_element_type=jnp.float32) + bc1_ref[...]
    h = jnp.where(h >= 0, h, pa_ref[...] * h)
    p = (jnp.dot(h, wc2_ref[...], preferred_element_type=jnp.float32)
         + bc2_ref[...])
    p3 = p.reshape(GB, N, 2)

    ids = knn_ref[...]
    iota_n = jax.lax.broadcasted_iota(jnp.int32, (GB, N), 1)
    edges = []
    for k in range(K):
        mask = (iota_n == ids[:, k][:, None]).astype(jnp.float32)
        edges.append(jnp.sum(mask[:, :, None] * p3, axis=1))
    out_ref[...] = jnp.stack(edges, axis=1).reshape(GB * K, 2)


def kernel(inputs, node_feats, A, knn_inds, conv_w, conv_b,
           w1, b1, w2, b2, w3, b3, w4, b4, wc1, bc1, prelu_a, wc2, bc2):
    stats = pl.pallas_call(
        _stats_kernel,
        grid=(G // SB,),
        in_specs=[pl.BlockSpec((SB, N, D_IN), lambda i: (i, 0, 0))],
        out_specs=pl.BlockSpec((2, D_IN), lambda i: (0, 0)),
        out_shape=jax.ShapeDtypeStruct((2, D_IN), jnp.float32),
    )(node_feats)

    w1b = w1.astype(jnp.bfloat16)
    w2b = w2.astype(jnp.bfloat16)
    w3b = w3.astype(jnp.bfloat16)
    w4b = w4.astype(jnp.bfloat16)
    x2 = inputs.reshape(C_IN, H * W)
    const = lambda shape: pl.BlockSpec(shape, lambda i: tuple(0 for _ in shape))
    gcn_pred, pred = pl.pallas_call(
        _gcn_kernel,
        grid=(G // GB,),
        in_specs=[
            pl.BlockSpec((GB, N, D_IN), lambda i: (i, 0, 0)),
            pl.BlockSpec((GB, N, N), lambda i: (i, 0, 0)),
            pl.BlockSpec((GB, K), lambda i: (i, 0)),
            const((2, D_IN)),
            const((2 * D_IN, 512)), const((512,)),
            const((1024, 256)), const((256,)),
            const((512, 128)), const((128,)),
            const((256, 64)), const((64,)),
            const((64, 32)), const((32,)), const((32,)),
            const((32, 2)), const((2,)),
            pl.BlockSpec((C_IN, HWB), lambda i: (0, i)),
            const((C_OUT, C_IN)),
            const((C_OUT, 1)),
        ],
        out_specs=[
            pl.BlockSpec((GB * K, 2), lambda i: (i, 0)),
            pl.BlockSpec((C_OUT, HWB), lambda i: (0, i)),
        ],
        out_shape=[
            jax.ShapeDtypeStruct((G * K, 2), jnp.float32),
            jax.ShapeDtypeStruct((C_OUT, H * W), jnp.float32),
        ],
    )(node_feats, A, knn_inds, stats,
      w1b, b1, w2b, b2, w3b, b3, w4b, b4, wc1, bc1, prelu_a, wc2, bc2,
      x2, conv_w, conv_b.reshape(C_OUT, 1))
    pred_maps = pred.reshape(1, C_OUT, H, W)

    return (pred_maps, gcn_pred)
